# baseline (speedup 1.0000x reference)
import dataclasses
import functools

import jax
import jax.numpy as jnp
from jax import lax
from jax.experimental import pallas as pl
from jax.experimental.pallas import tpu as pltpu
from jax.experimental.pallas import tpu_sc as plsc

_B = 16384
_D = 64
_NC = 2
_NS = 16
_L = 16
_NW = _NC * _NS
_BPW = _B // _NW
_CH = 128
_NCH = _BPW // _CH


def _sqrt16(x):
    i = plsc.bitcast(x, jnp.int32)
    i = (i >> 1) + jnp.int32(0x1FBD1DF6)
    y = plsc.bitcast(i, jnp.float32)
    for _ in range(3):
        y = 0.5 * (y + x / y)
    return y


def _body(ent_hbm, rel_hbm, hidx_hbm, tidx_hbm, lidx_hbm, out_hbm,
          hidx_v, tidx_v, lidx_v, h_rows, t_rows, r_rows, s_v, out_v, sem):
    wid = lax.axis_index("s") * _NC + lax.axis_index("c")
    base = wid * _BPW

    pltpu.sync_copy(hidx_hbm.at[wid], hidx_v)
    pltpu.sync_copy(tidx_hbm.at[wid], tidx_v)
    pltpu.sync_copy(lidx_hbm.at[wid], lidx_v)

    copies = []
    for c in range(_NCH):
        sl = pl.ds(c * _CH, _CH)
        copies.append(pltpu.async_copy(ent_hbm.at[hidx_v.at[c]], h_rows.at[sl], sem))
        copies.append(pltpu.async_copy(ent_hbm.at[tidx_v.at[c]], t_rows.at[sl], sem))
        copies.append(pltpu.async_copy(rel_hbm.at[lidx_v.at[c]], r_rows.at[sl], sem))
    for cp in copies:
        cp.wait()

    @pl.loop(0, _BPW)
    def _(j):
        acc = None
        for k in range(_D // _L):
            sl = pl.ds(k * _L, _L)
            d = h_rows[j, sl] + r_rows[j, sl] - t_rows[j, sl]
            acc = d * d if acc is None else acc + d * d
        s_v[j, :] = acc

    lane = lax.iota(jnp.int32, _L)

    @pl.loop(0, _BPW // _L)
    def _(g):
        rows = lane + g * _L
        acc = plsc.load_gather(s_v, [rows, jnp.zeros((_L,), jnp.int32)])
        for p in range(1, _L):
            acc = acc + plsc.load_gather(s_v, [rows, jnp.full((_L,), p, jnp.int32)])
        out_v[pl.ds(g * _L, _L)] = _sqrt16(acc)

    pltpu.sync_copy(out_v, out_hbm.at[pl.ds(base, _BPW)])


@jax.jit
def _transe_sc(entity_emb, relation_emb, hidx, tidx, lidx):
    mesh = plsc.VectorSubcoreMesh(core_axis_name="c", subcore_axis_name="s")
    cp = pltpu.CompilerParams(
        needs_layout_passes=False, use_tc_tiling_on_sc=False
    )
    k = pl.kernel(
        _body,
        out_type=jax.ShapeDtypeStruct((_B,), jnp.float32),
        mesh=mesh,
        scratch_types=[
            pltpu.VMEM((_NCH, _CH), jnp.int32),
            pltpu.VMEM((_NCH, _CH), jnp.int32),
            pltpu.VMEM((_NCH, _CH), jnp.int32),
            pltpu.VMEM((_BPW, _D), jnp.float32),
            pltpu.VMEM((_BPW, _D), jnp.float32),
            pltpu.VMEM((_BPW, _D), jnp.float32),
            pltpu.VMEM((_BPW, _L), jnp.float32),
            pltpu.VMEM((_BPW,), jnp.float32),
            pltpu.SemaphoreType.DMA,
        ],
        compiler_params=cp,
    )
    return k(entity_emb, relation_emb, hidx, tidx, lidx)


def kernel(head, tail, label, entity_emb, relation_emb):
    hidx = head.astype(jnp.int32).reshape(_NW, _NCH, _CH)
    tidx = tail.astype(jnp.int32).reshape(_NW, _NCH, _CH)
    lidx = label.astype(jnp.int32).reshape(_NW, _NCH, _CH)
    return _transe_sc(entity_emb, relation_emb, hidx, tidx, lidx)
